# baseline (speedup 1.0000x reference)
import functools

import jax
import jax.numpy as jnp
from jax import lax
from jax.experimental import pallas as pl
from jax.experimental.pallas import tpu as pltpu

_BN_EPS = 1e-5


def _shifted(h, s):
    if s == 0:
        return h
    C, T = h.shape
    return jnp.concatenate([jnp.zeros((C, s), h.dtype), h[:, :T - s]], axis=1)


def _gcn1d_kernel(x_ref, wa_ref, wb_ref, w0a_ref, ow0_ref, g_ref, b_ref,
                  ow_ref, o_ref, *, n_blocks, n_channels):
    C = n_channels

    x8 = x_ref[0]
    h = None
    for blk in range(n_blocks):
        dil = 2 ** blk
        if blk == 0:
            tap = jnp.concatenate(
                [_shifted(x8, 2), _shifted(x8, 1), x8], axis=0)
            wa, wb, res_in = w0a_ref[...], None, x8
        else:
            tap = jnp.concatenate(
                [_shifted(h, 2 * dil), _shifted(h, dil), h], axis=0)
            wa, wb, res_in = wa_ref[blk], wb_ref[blk], h
        ya = jnp.dot(wa, tap, preferred_element_type=jnp.float32)
        if blk == 0:
            yb = ow0_ref[...] * x8[:1, :].astype(jnp.float32)
        else:
            yb = jnp.dot(wb, res_in, preferred_element_type=jnp.float32)
        z = ya * g_ref[0, :, blk:blk + 1] + b_ref[0, :, blk:blk + 1]
        sig = 0.5 + 0.5 * jnp.tanh(0.5 * z[C:])
        hf = jnp.tanh(z[:C]) * sig + yb
        if blk < n_blocks - 1:
            h = hf.astype(jnp.bfloat16)
        else:
            o = jnp.sum(hf * ow_ref[...], axis=0, keepdims=True)
            o_ref[0, :, :] = jnp.tanh(o)


def kernel(x, cond, w_stack, out_w, adaptor_w, adaptor_b, bn_mean, bn_var):
    B, in_ch, T = x.shape
    n_blocks, C3, KC = w_stack.shape
    C = C3 // 3
    K = KC // C
    out_ch = out_w.shape[0]

    proj = jnp.einsum("bd,nod->bno", cond, adaptor_w,
                      precision=lax.Precision.HIGHEST) + adaptor_b
    g, b = proj[..., :2 * C], proj[..., 2 * C:]
    inv_std = lax.rsqrt(bn_var + _BN_EPS)
    gamma = jnp.transpose(g * inv_std, (0, 2, 1))
    beta = jnp.transpose(b - g * (bn_mean * inv_std), (0, 2, 1))

    x8 = jnp.pad(x.astype(jnp.bfloat16), ((0, 0), (0, 8 - in_ch), (0, 0)))

    wa = w_stack[:, :2 * C, :].astype(jnp.bfloat16)
    wb = w_stack[:, 2 * C:, (K - 1) * C:].astype(jnp.bfloat16)
    w0 = w_stack[0].reshape(C3, K, C)[:, :, :8].reshape(C3, K * 8)
    w0a = w0[:2 * C].astype(jnp.bfloat16)
    w0b = w0[2 * C:, (K - 1) * 8:(K - 1) * 8 + 1].astype(jnp.float32)
    ow_t = out_w.astype(jnp.float32).T

    body = functools.partial(_gcn1d_kernel, n_blocks=n_blocks, n_channels=C)
    return pl.pallas_call(
        body,
        out_shape=jax.ShapeDtypeStruct((B, out_ch, T), x.dtype),
        grid_spec=pltpu.PrefetchScalarGridSpec(
            num_scalar_prefetch=0,
            grid=(B,),
            in_specs=[
                pl.BlockSpec((1, 8, T), lambda bb: (bb, 0, 0)),
                pl.BlockSpec((n_blocks, 2 * C, KC), lambda bb: (0, 0, 0)),
                pl.BlockSpec((n_blocks, C, C), lambda bb: (0, 0, 0)),
                pl.BlockSpec((2 * C, K * 8), lambda bb: (0, 0)),
                pl.BlockSpec((C, 1), lambda bb: (0, 0)),
                pl.BlockSpec((1, 2 * C, n_blocks), lambda bb: (bb, 0, 0)),
                pl.BlockSpec((1, 2 * C, n_blocks), lambda bb: (bb, 0, 0)),
                pl.BlockSpec((C, out_ch), lambda bb: (0, 0)),
            ],
            out_specs=pl.BlockSpec((1, out_ch, T), lambda bb: (bb, 0, 0)),
        ),
        compiler_params=pltpu.CompilerParams(
            dimension_semantics=("parallel",)),
    )(x8, wa, wb, w0a, w0b, gamma, beta, ow_t)


# speedup vs baseline: 1.0065x; 1.0065x over previous
import functools

import jax
import jax.numpy as jnp
from jax import lax
from jax.experimental import pallas as pl
from jax.experimental.pallas import tpu as pltpu

_BN_EPS = 1e-5


def _shifted(h, s):
    if s == 0:
        return h
    C, T = h.shape
    return jnp.concatenate([jnp.zeros((C, s), h.dtype), h[:, :T - s]], axis=1)


def _gcn1d_kernel(x_ref, wa_ref, wb_ref, w0a_ref, w0b_ref, g_ref, b_ref,
                  ow_ref, o_ref, *, n_blocks, n_channels):
    C = n_channels

    x8 = x_ref[0]
    h = None
    for blk in range(n_blocks):
        dil = 2 ** blk
        if blk == 0:
            tap = jnp.concatenate(
                [_shifted(x8, 2), _shifted(x8, 1), x8], axis=0)
            wa, wb, res_in = w0a_ref[...], w0b_ref[...], x8
        else:
            tap = jnp.concatenate(
                [_shifted(h, 2 * dil), _shifted(h, dil), h], axis=0)
            wa, wb, res_in = wa_ref[blk], wb_ref[blk], h
        ya = jnp.dot(wa, tap, preferred_element_type=jnp.float32)
        yb = jnp.dot(wb, res_in, preferred_element_type=jnp.float32)
        z = ya * g_ref[0, :, blk:blk + 1] + b_ref[0, :, blk:blk + 1]
        sig = 0.5 + 0.5 * jnp.tanh(0.5 * z[C:])
        hf = jnp.tanh(z[:C]) * sig + yb
        if blk < n_blocks - 1:
            h = hf.astype(jnp.bfloat16)
        else:
            o = jnp.sum(hf * ow_ref[...], axis=0, keepdims=True)
            o_ref[0, :, :] = jnp.tanh(o)


def kernel(x, cond, w_stack, out_w, adaptor_w, adaptor_b, bn_mean, bn_var):
    B, in_ch, T = x.shape
    n_blocks, C3, KC = w_stack.shape
    C = C3 // 3
    K = KC // C
    out_ch = out_w.shape[0]

    proj = jnp.einsum("bd,nod->bno", cond, adaptor_w,
                      precision=lax.Precision.HIGHEST) + adaptor_b
    g, b = proj[..., :2 * C], proj[..., 2 * C:]
    inv_std = lax.rsqrt(bn_var + _BN_EPS)
    gamma = jnp.transpose(g * inv_std, (0, 2, 1))
    beta = jnp.transpose(b - g * (bn_mean * inv_std), (0, 2, 1))

    x8 = jnp.pad(x.astype(jnp.bfloat16), ((0, 0), (0, 8 - in_ch), (0, 0)))

    wa = w_stack[:, :2 * C, :].astype(jnp.bfloat16)
    wb = w_stack[:, 2 * C:, (K - 1) * C:].astype(jnp.bfloat16)
    w0 = w_stack[0].reshape(C3, K, C)[:, :, :8].reshape(C3, K * 8)
    w0a = w0[:2 * C].astype(jnp.bfloat16)
    w0b = w0[2 * C:, (K - 1) * 8:].astype(jnp.bfloat16)
    ow_t = out_w.astype(jnp.float32).T

    body = functools.partial(_gcn1d_kernel, n_blocks=n_blocks, n_channels=C)
    return pl.pallas_call(
        body,
        out_shape=jax.ShapeDtypeStruct((B, out_ch, T), x.dtype),
        grid_spec=pltpu.PrefetchScalarGridSpec(
            num_scalar_prefetch=0,
            grid=(B,),
            in_specs=[
                pl.BlockSpec((1, 8, T), lambda bb: (bb, 0, 0)),
                pl.BlockSpec((n_blocks, 2 * C, KC), lambda bb: (0, 0, 0)),
                pl.BlockSpec((n_blocks, C, C), lambda bb: (0, 0, 0)),
                pl.BlockSpec((2 * C, K * 8), lambda bb: (0, 0)),
                pl.BlockSpec((C, 8), lambda bb: (0, 0)),
                pl.BlockSpec((1, 2 * C, n_blocks), lambda bb: (bb, 0, 0)),
                pl.BlockSpec((1, 2 * C, n_blocks), lambda bb: (bb, 0, 0)),
                pl.BlockSpec((C, out_ch), lambda bb: (0, 0)),
            ],
            out_specs=pl.BlockSpec((1, out_ch, T), lambda bb: (bb, 0, 0)),
        ),
        compiler_params=pltpu.CompilerParams(
            dimension_semantics=("parallel",)),
    )(x8, wa, wb, w0a, w0b, gamma, beta, ow_t)
